# Initial kernel scaffold; baseline (speedup 1.0000x reference)
import jax
import jax.numpy as jnp
from jax.experimental import pallas as pl


def kernel(data, embedding):
    raise NotImplementedError("write your pallas kernel here")


# speedup vs baseline: 4.4413x; 4.4413x over previous
import functools

import jax
import jax.numpy as jnp
from jax import lax
from jax.experimental import pallas as pl
from jax.experimental.pallas import tpu as pltpu
from jax.experimental.pallas import tpu_sc as plsc

_N = 10000
_E = 320000
_D = 128
_NW = 32
_EPW = _E // _NW
_C = 80
_NCH = _EPW // _C
_G = _C // 16


@functools.partial(
    pl.kernel,
    mesh=plsc.VectorSubcoreMesh(core_axis_name="c", subcore_axis_name="s"),
    out_type=jax.ShapeDtypeStruct((_E,), jnp.float32),
    compiler_params=pltpu.CompilerParams(needs_layout_passes=False),
    scratch_types=[
        pltpu.VMEM((_EPW,), jnp.int32),
        pltpu.VMEM((_EPW,), jnp.int32),
        pltpu.VMEM((_C, _D), jnp.float32),
        pltpu.VMEM((_C, _D), jnp.float32),
        pltpu.VMEM((256,), jnp.float32),
        pltpu.VMEM((_C,), jnp.float32),
        pltpu.SemaphoreType.DMA,
        pltpu.SemaphoreType.DMA,
    ],
)
def _edge_dot(d0_hbm, d1_hbm, table_hbm, out_hbm,
              idx0_v, idx1_v, rows0_v, rows1_v, tbuf_v, out_v, sem0, sem1):
    cid = lax.axis_index("c")
    sid = lax.axis_index("s")
    wid = sid * 2 + cid
    base_w = wid * _EPW

    pltpu.sync_copy(d0_hbm.at[pl.ds(base_w, _EPW)], idx0_v)
    pltpu.sync_copy(d1_hbm.at[pl.ds(base_w, _EPW)], idx1_v)

    def chunk_body(t, carry):
        off = t * _C
        cp0 = pltpu.async_copy(table_hbm.at[idx0_v.at[pl.ds(off, _C)]],
                               rows0_v, sem0)
        cp1 = pltpu.async_copy(table_hbm.at[idx1_v.at[pl.ds(off, _C)]],
                               rows1_v, sem1)
        cp0.wait()
        cp1.wait()

        def group_body(g, carry2):
            for e16 in range(16):
                e = g * 16 + e16
                acc = jnp.zeros((16,), jnp.float32)
                for k in range(_D // 16):
                    a = rows0_v[e, pl.ds(k * 16, 16)]
                    b = rows1_v[e, pl.ds(k * 16, 16)]
                    acc = acc + a * b
                tbuf_v[pl.ds(e16 * 16, 16)] = acc
            ids = lax.iota(jnp.int32, 16) * 16
            out = jnp.zeros((16,), jnp.float32)
            for l in range(16):
                out = out + plsc.load_gather(tbuf_v, [ids + l])
            out_v[pl.ds(g * 16, 16)] = out
            return carry2

        lax.fori_loop(0, _G, group_body, 0)
        pltpu.sync_copy(out_v, out_hbm.at[pl.ds(base_w + off, _C)])
        return carry

    lax.fori_loop(0, _NCH, chunk_body, 0)


def kernel(data, embedding):
    return _edge_dot(data[0], data[1], embedding)


# speedup vs baseline: 7.6326x; 1.7186x over previous
import functools

import jax
import jax.numpy as jnp
from jax import lax
from jax.experimental import pallas as pl
from jax.experimental.pallas import tpu as pltpu
from jax.experimental.pallas import tpu_sc as plsc

_N = 10000
_E = 320000
_D = 128
_NW = 32
_EPW = _E // _NW
_C = 80
_NCH = _EPW // _C
_G = _C // 16
_NBUF = 4


@functools.partial(
    pl.kernel,
    mesh=plsc.VectorSubcoreMesh(core_axis_name="c", subcore_axis_name="s"),
    out_type=jax.ShapeDtypeStruct((_E,), jnp.float32),
    compiler_params=pltpu.CompilerParams(needs_layout_passes=False),
    scratch_types=[
        pltpu.VMEM((_EPW,), jnp.int32),
        pltpu.VMEM((_EPW,), jnp.int32),
        [pltpu.VMEM((_C, _D), jnp.float32) for _ in range(_NBUF)],
        [pltpu.VMEM((_C, _D), jnp.float32) for _ in range(_NBUF)],
        pltpu.VMEM((256,), jnp.float32),
        pltpu.VMEM((_EPW,), jnp.float32),
        [pltpu.SemaphoreType.DMA for _ in range(_NBUF)],
    ],
)
def _edge_dot(d0_hbm, d1_hbm, table_hbm, out_hbm,
              idx0_v, idx1_v, rows0_bufs, rows1_bufs, tbuf_v, out_v, sems):
    cid = lax.axis_index("c")
    sid = lax.axis_index("s")
    wid = sid * 2 + cid
    base_w = wid * _EPW

    pltpu.sync_copy(d0_hbm.at[pl.ds(base_w, _EPW)], idx0_v)
    pltpu.sync_copy(d1_hbm.at[pl.ds(base_w, _EPW)], idx1_v)

    def issue(t, b):
        off = t * _C
        pltpu.async_copy(table_hbm.at[idx0_v.at[pl.ds(off, _C)]],
                         rows0_bufs[b], sems[b])
        pltpu.async_copy(table_hbm.at[idx1_v.at[pl.ds(off, _C)]],
                         rows1_bufs[b], sems[b])

    def drain(b):
        pltpu.make_async_copy(table_hbm.at[pl.ds(0, _C)],
                              rows0_bufs[b], sems[b]).wait()
        pltpu.make_async_copy(table_hbm.at[pl.ds(0, _C)],
                              rows1_bufs[b], sems[b]).wait()

    def compute(t, b):
        r0, r1 = rows0_bufs[b], rows1_bufs[b]

        def group_body(g, carry):
            for e16 in range(16):
                e = g * 16 + e16
                acc = jnp.zeros((16,), jnp.float32)
                for k in range(_D // 16):
                    acc = acc + r0[e, pl.ds(k * 16, 16)] * r1[e, pl.ds(k * 16, 16)]
                tbuf_v[pl.ds(e16 * 16, 16)] = acc
            ids = lax.iota(jnp.int32, 16) * 16
            o = jnp.zeros((16,), jnp.float32)
            for l in range(16):
                o = o + plsc.load_gather(tbuf_v, [ids + l])
            out_v[pl.ds(t * _C + g * 16, 16)] = o
            return carry

        lax.fori_loop(0, _G, group_body, 0)

    for b in range(_NBUF):
        issue(b, b)

    def loop_body(tt, carry):
        for b in range(_NBUF):
            t = tt * _NBUF + b
            drain(b)
            compute(t, b)

            @pl.when(t + _NBUF < _NCH)
            def _():
                issue(t + _NBUF, b)

        return carry

    lax.fori_loop(0, _NCH // _NBUF, loop_body, 0)

    for t in range((_NCH // _NBUF) * _NBUF, _NCH):
        b = t % _NBUF
        drain(b)
        compute(t, b)

    pltpu.sync_copy(out_v, out_hbm.at[pl.ds(base_w, _EPW)])


def kernel(data, embedding):
    return _edge_dot(data[0], data[1], embedding)
